# Initial kernel scaffold; baseline (speedup 1.0000x reference)
import jax
import jax.numpy as jnp
from jax.experimental import pallas as pl


def kernel(x, edge_index, W0, att_src0, att_dst0, b0, W1, att_src1, att_dst1, b1):
    raise NotImplementedError("write your pallas kernel here")


# speedup vs baseline: 24.4028x; 24.4028x over previous
import functools

import jax
import jax.numpy as jnp
from jax import lax
from jax.experimental import pallas as pl
from jax.experimental.pallas import tpu as pltpu
from jax.experimental.pallas import tpu_sc as plsc

N = 10000
E = 320000
F_IN = 128
HID = 64
OUT = 32

NP = 10240
NC, NS, L = 2, 16, 16
E2 = E // NC
TE = E2 // NS
CH = 2000
NEG = 0.2
NB = 2048


def _lrelu(v):
    return jnp.where(v >= 0, v, NEG * v)


def _make_edge_kernel(C):
    CPT = C // NS
    mesh = plsc.VectorSubcoreMesh(
        core_axis_name="c", subcore_axis_name="s", num_cores=NC, num_subcores=NS)

    @functools.partial(
        pl.kernel,
        out_type=[
            jax.ShapeDtypeStruct((NC, C, NP), jnp.float32),
            jax.ShapeDtypeStruct((NC, NS, NP), jnp.float32),
        ],
        mesh=mesh,
        compiler_params=pltpu.CompilerParams(needs_layout_passes=False),
        scratch_types=[
            pltpu.VMEM_SHARED((E2,), jnp.float32),
            pltpu.VMEM((NP,), jnp.float32),
            pltpu.VMEM((NP,), jnp.float32),
            pltpu.VMEM((NP,), jnp.float32),
            pltpu.VMEM((CPT, NP), jnp.float32),
            pltpu.VMEM((CPT, NP), jnp.float32),
            pltpu.VMEM((CH,), jnp.int32),
            pltpu.VMEM((CH,), jnp.int32),
            pltpu.VMEM((CH,), jnp.float32),
        ],
    )
    def edge_kernel(src_h, dst_h, asrc_h, adst_h, hT_h, S_h, den_h,
                    w_sh, asrc_l, adst_l, den_l, h_l, s_l, src_b, dst_b, w_b):
        c = lax.axis_index("c")
        s = lax.axis_index("s")

        pltpu.sync_copy(asrc_h, asrc_l)
        pltpu.sync_copy(adst_h, adst_l)
        pltpu.sync_copy(hT_h.at[pl.ds(s * CPT, CPT)], h_l)

        def zero_den(i, _):
            den_l[pl.ds(i * L, L)] = jnp.zeros((L,), jnp.float32)
            return 0
        lax.fori_loop(0, NP // L, zero_den, 0)

        def zero_s(i, _):
            for cc in range(CPT):
                s_l[cc, pl.ds(i * L, L)] = jnp.zeros((L,), jnp.float32)
            return 0
        lax.fori_loop(0, NP // L, zero_s, 0)

        eb = c * E2 + s * TE

        def p1_chunk(k, _):
            off = eb + k * CH
            pltpu.sync_copy(src_h.at[pl.ds(off, CH)], src_b)
            pltpu.sync_copy(dst_h.at[pl.ds(off, CH)], dst_b)

            def p1_body(i, _):
                sv = src_b[pl.ds(i * L, L)]
                dv = dst_b[pl.ds(i * L, L)]
                av = plsc.load_gather(asrc_l, [sv]) + plsc.load_gather(adst_l, [dv])
                w = jnp.exp(_lrelu(av))
                w_b[pl.ds(i * L, L)] = w
                plsc.addupdate_scatter(den_l, [dv], w)
                return 0
            lax.fori_loop(0, CH // L, p1_body, 0)
            pltpu.sync_copy(w_b, w_sh.at[pl.ds(s * TE + k * CH, CH)])
            return 0
        lax.fori_loop(0, TE // CH, p1_chunk, 0)

        pltpu.sync_copy(den_l, den_h.at[c, s])
        plsc.subcore_barrier()

        def p2_chunk(k, _):
            off = c * E2 + k * CH
            pltpu.sync_copy(src_h.at[pl.ds(off, CH)], src_b)
            pltpu.sync_copy(dst_h.at[pl.ds(off, CH)], dst_b)
            pltpu.sync_copy(w_sh.at[pl.ds(k * CH, CH)], w_b)

            def p2_body(i, _):
                sv = src_b[pl.ds(i * L, L)]
                dv = dst_b[pl.ds(i * L, L)]
                wv = w_b[pl.ds(i * L, L)]
                for cc in range(CPT):
                    ccv = jnp.full((L,), cc, jnp.int32)
                    hv = plsc.load_gather(h_l, [ccv, sv])
                    plsc.addupdate_scatter(s_l, [ccv, dv], hv * wv)
                return 0
            lax.fori_loop(0, CH // L, p2_body, 0)
            return 0
        lax.fori_loop(0, E2 // CH, p2_chunk, 0)

        pltpu.sync_copy(s_l, S_h.at[c, pl.ds(s * CPT, CPT)])

    return edge_kernel


_edge_kernel0 = _make_edge_kernel(HID)
_edge_kernel1 = _make_edge_kernel(OUT)


def _tc1_body(xT_ref, w0T_ref, as_ref, ad_ref,
              hT_ref, aso_ref, ado_ref, ws_ref):
    hT = jnp.dot(w0T_ref[...], xT_ref[...], preferred_element_type=jnp.float32)
    hT_ref[...] = hT
    a_s = jnp.dot(as_ref[...], hT, preferred_element_type=jnp.float32)
    a_d = jnp.dot(ad_ref[...], hT, preferred_element_type=jnp.float32)
    aso_ref[...] = a_s
    ado_ref[...] = a_d
    ws_ref[...] = jnp.exp(_lrelu(a_s + a_d))


def _tc2_body(S_ref, den_ref, ws_ref, hT_ref, b0_ref, w1T_ref, as1_ref, ad1_ref,
              h1T_ref, aso_ref, ado_ref, wso_ref):
    ws = ws_ref[...]
    den = jnp.sum(den_ref[...], axis=0, keepdims=True) + ws + 1e-16
    Sb = S_ref[0:HID, :] + S_ref[HID:2 * HID, :] + ws * hT_ref[...]
    x1 = Sb / den + b0_ref[...]
    x1 = jnp.where(x1 > 0, x1, jnp.exp(x1) - 1.0)
    h1T = jnp.dot(w1T_ref[...], x1, preferred_element_type=jnp.float32)
    h1T_ref[...] = h1T
    a_s = jnp.dot(as1_ref[...], h1T, preferred_element_type=jnp.float32)
    a_d = jnp.dot(ad1_ref[...], h1T, preferred_element_type=jnp.float32)
    aso_ref[...] = a_s
    ado_ref[...] = a_d
    wso_ref[...] = jnp.exp(_lrelu(a_s + a_d))


def _tc3_body(S_ref, den_ref, ws_ref, hT_ref, b1_ref, o_ref):
    ws = ws_ref[...]
    den = jnp.sum(den_ref[...], axis=0, keepdims=True) + ws + 1e-16
    Ob = (S_ref[0:OUT, :] + S_ref[OUT:2 * OUT, :] + ws * hT_ref[...]) / den \
        + b1_ref[...]
    m = jnp.max(Ob, axis=0, keepdims=True)
    lse = m + jnp.log(jnp.sum(jnp.exp(Ob - m), axis=0, keepdims=True))
    o_ref[...] = Ob - lse


def _row_spec(rows):
    return pl.BlockSpec((rows, NB), lambda i: (0, i))


def _fix_spec(shape):
    return pl.BlockSpec(shape, lambda i: tuple(0 for _ in shape))


_GRID = NP // NB

_tc1 = pl.pallas_call(
    _tc1_body,
    grid=(_GRID,),
    in_specs=[_row_spec(F_IN), _fix_spec((HID, F_IN)), _fix_spec((1, HID)),
              _fix_spec((1, HID))],
    out_specs=[_row_spec(HID), _row_spec(1), _row_spec(1), _row_spec(1)],
    out_shape=[jax.ShapeDtypeStruct((HID, NP), jnp.float32),
               jax.ShapeDtypeStruct((1, NP), jnp.float32),
               jax.ShapeDtypeStruct((1, NP), jnp.float32),
               jax.ShapeDtypeStruct((1, NP), jnp.float32)],
)

_tc2 = pl.pallas_call(
    _tc2_body,
    grid=(_GRID,),
    in_specs=[_row_spec(2 * HID), _row_spec(NC * NS), _row_spec(1),
              _row_spec(HID), _fix_spec((HID, NB)), _fix_spec((OUT, HID)),
              _fix_spec((1, OUT)), _fix_spec((1, OUT))],
    out_specs=[_row_spec(OUT), _row_spec(1), _row_spec(1), _row_spec(1)],
    out_shape=[jax.ShapeDtypeStruct((OUT, NP), jnp.float32),
               jax.ShapeDtypeStruct((1, NP), jnp.float32),
               jax.ShapeDtypeStruct((1, NP), jnp.float32),
               jax.ShapeDtypeStruct((1, NP), jnp.float32)],
)

_tc3 = pl.pallas_call(
    _tc3_body,
    grid=(_GRID,),
    in_specs=[_row_spec(2 * OUT), _row_spec(NC * NS), _row_spec(1),
              _row_spec(OUT), _fix_spec((OUT, NB))],
    out_specs=_row_spec(OUT),
    out_shape=jax.ShapeDtypeStruct((OUT, NP), jnp.float32),
)


def kernel(x, edge_index, W0, att_src0, att_dst0, b0, W1, att_src1, att_dst1, b1):
    xT = jnp.zeros((F_IN, NP), jnp.float32).at[:, :N].set(x.T)
    src = edge_index[0]
    dst = edge_index[1]
    as0 = att_src0.reshape(1, HID)
    ad0 = att_dst0.reshape(1, HID)
    as1 = att_src1.reshape(1, OUT)
    ad1 = att_dst1.reshape(1, OUT)
    b0b = jnp.broadcast_to(b0[:, None], (HID, NB))
    b1b = jnp.broadcast_to(b1[:, None], (OUT, NB))

    hT0, a_s0, a_d0, ws0 = _tc1(xT, W0.T, as0, ad0)
    S0, den0 = _edge_kernel0(src, dst, a_s0.reshape(NP), a_d0.reshape(NP), hT0)
    h1T, a_s1, a_d1, ws1 = _tc2(
        S0.reshape(NC * HID, NP), den0.reshape(NC * NS, NP), ws0, hT0, b0b,
        W1.T, as1, ad1)
    S1, den1 = _edge_kernel1(src, dst, a_s1.reshape(NP), a_d1.reshape(NP), h1T)
    oT = _tc3(S1.reshape(NC * OUT, NP), den1.reshape(NC * NS, NP), ws1, h1T, b1b)
    return oT.T[:N, :]


# speedup vs baseline: 27.3326x; 1.1201x over previous
import functools

import jax
import jax.numpy as jnp
from jax import lax
from jax.experimental import pallas as pl
from jax.experimental.pallas import tpu as pltpu
from jax.experimental.pallas import tpu_sc as plsc

N = 10000
E = 320000
F_IN = 128
HID = 64
OUT = 32

NP = 10240
NC, NS, L = 2, 16, 16
E2 = E // NC
TE = E2 // NS
CH = 2000
NEG = 0.2
NB = 2048


def _lrelu(v):
    return jnp.where(v >= 0, v, NEG * v)


def _make_edge_kernel(C):
    CPT = C // NS
    NK = E2 // CH
    KPT = NK // NS
    mesh = plsc.VectorSubcoreMesh(
        core_axis_name="c", subcore_axis_name="s", num_cores=NC, num_subcores=NS)

    @functools.partial(
        pl.kernel,
        out_type=[
            jax.ShapeDtypeStruct((NC, C, NP), jnp.float32),
            jax.ShapeDtypeStruct((NC, NS, NP), jnp.float32),
        ],
        mesh=mesh,
        compiler_params=pltpu.CompilerParams(needs_layout_passes=False),
        scratch_types=[
            pltpu.VMEM((NP,), jnp.float32),
            pltpu.VMEM((NP,), jnp.float32),
            pltpu.VMEM((NP,), jnp.float32),
            pltpu.VMEM((CPT, NP), jnp.float32),
            pltpu.VMEM((CPT, NP), jnp.float32),
            pltpu.VMEM((CH,), jnp.int32),
            pltpu.VMEM((CH,), jnp.int32),
            pltpu.VMEM((CH,), jnp.int32),
            pltpu.VMEM((CH,), jnp.int32),
            pltpu.SemaphoreType.DMA,
            pltpu.SemaphoreType.DMA,
        ],
    )
    def edge_kernel(src_h, dst_h, asrc_h, adst_h, hT_h, S_h, den_h,
                    asrc_l, adst_l, den_l, h_l, s_l, src_b0, src_b1,
                    dst_b0, dst_b1, sem0, sem1):
        c = lax.axis_index("c")
        s = lax.axis_index("s")

        pltpu.sync_copy(asrc_h, asrc_l)
        pltpu.sync_copy(adst_h, adst_l)
        pltpu.sync_copy(hT_h.at[pl.ds(s * CPT, CPT)], h_l)

        def zero_all(i, _):
            den_l[pl.ds(i * L, L)] = jnp.zeros((L,), jnp.float32)
            for cc in range(CPT):
                s_l[cc, pl.ds(i * L, L)] = jnp.zeros((L,), jnp.float32)
            return 0
        lax.fori_loop(0, NP // L, zero_all, 0)

        ebase = c * E2
        bufs = ((src_b0, dst_b0, sem0), (src_b1, dst_b1, sem1))

        def fetch(k, b):
            off = ebase + k * CH
            sb, db, sem = bufs[b]
            cp_s = pltpu.make_async_copy(src_h.at[pl.ds(off, CH)], sb, sem)
            cp_d = pltpu.make_async_copy(dst_h.at[pl.ds(off, CH)], db, sem)
            return cp_s, cp_d

        def start_fetch(k, b):
            cp_s, cp_d = fetch(k, b)
            cp_s.start()
            cp_d.start()

        def wait_fetch(k, b):
            cp_s, cp_d = fetch(k, b)
            cp_s.wait()
            cp_d.wait()

        def make_inner(b, with_den):
            sb, db, _ = bufs[b]

            def inner(i, _):
                sv = sb[pl.ds(i * L, L)]
                dv = db[pl.ds(i * L, L)]
                av = plsc.load_gather(asrc_l, [sv]) + plsc.load_gather(adst_l, [dv])
                wv = jnp.exp(_lrelu(av))
                if with_den:
                    plsc.addupdate_scatter(den_l, [dv], wv)
                for cc in range(CPT):
                    ccv = jnp.full((L,), cc, jnp.int32)
                    hv = plsc.load_gather(h_l, [ccv, sv])
                    plsc.addupdate_scatter(s_l, [ccv, dv], hv * wv)
                return 0
            return inner

        def process(k, b):
            mine = jnp.logical_and(k >= s * KPT, k < (s + 1) * KPT)
            lax.cond(
                mine,
                lambda: lax.fori_loop(0, CH // L, make_inner(b, True), 0),
                lambda: lax.fori_loop(0, CH // L, make_inner(b, False), 0),
            )

        start_fetch(0, 0)

        def chunk_pair(kk, _):
            k0 = 2 * kk
            start_fetch(k0 + 1, 1)
            wait_fetch(k0, 0)
            process(k0, 0)

            @pl.when(k0 + 2 < NK)
            def _():
                start_fetch(k0 + 2, 0)

            wait_fetch(k0 + 1, 1)
            process(k0 + 1, 1)
            return 0
        lax.fori_loop(0, NK // 2, chunk_pair, 0)

        pltpu.sync_copy(den_l, den_h.at[c, s])
        pltpu.sync_copy(s_l, S_h.at[c, pl.ds(s * CPT, CPT)])

    return edge_kernel


_edge_kernel0 = _make_edge_kernel(HID)
_edge_kernel1 = _make_edge_kernel(OUT)


def _tc1_body(xT_ref, w0T_ref, as_ref, ad_ref,
              hT_ref, aso_ref, ado_ref, ws_ref):
    hT = jnp.dot(w0T_ref[...], xT_ref[...], preferred_element_type=jnp.float32)
    hT_ref[...] = hT
    a_s = jnp.dot(as_ref[...], hT, preferred_element_type=jnp.float32)
    a_d = jnp.dot(ad_ref[...], hT, preferred_element_type=jnp.float32)
    aso_ref[...] = a_s
    ado_ref[...] = a_d
    ws_ref[...] = jnp.exp(_lrelu(a_s + a_d))


def _tc2_body(S_ref, den_ref, ws_ref, hT_ref, b0_ref, w1T_ref, as1_ref, ad1_ref,
              h1T_ref, aso_ref, ado_ref, wso_ref):
    ws = ws_ref[...]
    den = jnp.sum(den_ref[...], axis=0, keepdims=True) + ws + 1e-16
    Sb = S_ref[0:HID, :] + S_ref[HID:2 * HID, :] + ws * hT_ref[...]
    x1 = Sb / den + b0_ref[...]
    x1 = jnp.where(x1 > 0, x1, jnp.exp(x1) - 1.0)
    h1T = jnp.dot(w1T_ref[...], x1, preferred_element_type=jnp.float32)
    h1T_ref[...] = h1T
    a_s = jnp.dot(as1_ref[...], h1T, preferred_element_type=jnp.float32)
    a_d = jnp.dot(ad1_ref[...], h1T, preferred_element_type=jnp.float32)
    aso_ref[...] = a_s
    ado_ref[...] = a_d
    wso_ref[...] = jnp.exp(_lrelu(a_s + a_d))


def _tc3_body(S_ref, den_ref, ws_ref, hT_ref, b1_ref, o_ref):
    ws = ws_ref[...]
    den = jnp.sum(den_ref[...], axis=0, keepdims=True) + ws + 1e-16
    Ob = (S_ref[0:OUT, :] + S_ref[OUT:2 * OUT, :] + ws * hT_ref[...]) / den \
        + b1_ref[...]
    m = jnp.max(Ob, axis=0, keepdims=True)
    lse = m + jnp.log(jnp.sum(jnp.exp(Ob - m), axis=0, keepdims=True))
    o_ref[...] = Ob - lse


def _row_spec(rows):
    return pl.BlockSpec((rows, NB), lambda i: (0, i))


def _fix_spec(shape):
    return pl.BlockSpec(shape, lambda i: tuple(0 for _ in shape))


_GRID = NP // NB

_tc1 = pl.pallas_call(
    _tc1_body,
    grid=(_GRID,),
    in_specs=[_row_spec(F_IN), _fix_spec((HID, F_IN)), _fix_spec((1, HID)),
              _fix_spec((1, HID))],
    out_specs=[_row_spec(HID), _row_spec(1), _row_spec(1), _row_spec(1)],
    out_shape=[jax.ShapeDtypeStruct((HID, NP), jnp.float32),
               jax.ShapeDtypeStruct((1, NP), jnp.float32),
               jax.ShapeDtypeStruct((1, NP), jnp.float32),
               jax.ShapeDtypeStruct((1, NP), jnp.float32)],
)

_tc2 = pl.pallas_call(
    _tc2_body,
    grid=(_GRID,),
    in_specs=[_row_spec(2 * HID), _row_spec(NC * NS), _row_spec(1),
              _row_spec(HID), _fix_spec((HID, NB)), _fix_spec((OUT, HID)),
              _fix_spec((1, OUT)), _fix_spec((1, OUT))],
    out_specs=[_row_spec(OUT), _row_spec(1), _row_spec(1), _row_spec(1)],
    out_shape=[jax.ShapeDtypeStruct((OUT, NP), jnp.float32),
               jax.ShapeDtypeStruct((1, NP), jnp.float32),
               jax.ShapeDtypeStruct((1, NP), jnp.float32),
               jax.ShapeDtypeStruct((1, NP), jnp.float32)],
)

_tc3 = pl.pallas_call(
    _tc3_body,
    grid=(_GRID,),
    in_specs=[_row_spec(2 * OUT), _row_spec(NC * NS), _row_spec(1),
              _row_spec(OUT), _fix_spec((OUT, NB))],
    out_specs=_row_spec(OUT),
    out_shape=jax.ShapeDtypeStruct((OUT, NP), jnp.float32),
)


def kernel(x, edge_index, W0, att_src0, att_dst0, b0, W1, att_src1, att_dst1, b1):
    xT = jnp.zeros((F_IN, NP), jnp.float32).at[:, :N].set(x.T)
    src = edge_index[0]
    dst = edge_index[1]
    as0 = att_src0.reshape(1, HID)
    ad0 = att_dst0.reshape(1, HID)
    as1 = att_src1.reshape(1, OUT)
    ad1 = att_dst1.reshape(1, OUT)
    b0b = jnp.broadcast_to(b0[:, None], (HID, NB))
    b1b = jnp.broadcast_to(b1[:, None], (OUT, NB))

    hT0, a_s0, a_d0, ws0 = _tc1(xT, W0.T, as0, ad0)
    S0, den0 = _edge_kernel0(src, dst, a_s0.reshape(NP), a_d0.reshape(NP), hT0)
    h1T, a_s1, a_d1, ws1 = _tc2(
        S0.reshape(NC * HID, NP), den0.reshape(NC * NS, NP), ws0, hT0, b0b,
        W1.T, as1, ad1)
    S1, den1 = _edge_kernel1(src, dst, a_s1.reshape(NP), a_d1.reshape(NP), h1T)
    oT = _tc3(S1.reshape(NC * OUT, NP), den1.reshape(NC * NS, NP), ws1, h1T, b1b)
    return oT.T[:N, :]


# speedup vs baseline: 71.7655x; 2.6256x over previous
import functools

import jax
import jax.numpy as jnp
from jax import lax
from jax.experimental import pallas as pl
from jax.experimental.pallas import tpu as pltpu
from jax.experimental.pallas import tpu_sc as plsc

N = 10000
E = 320000
F_IN = 128
HID = 64
OUT = 32

NP = 10240
NC, NS, L = 2, 16, 16
E2 = E // NC
TE = E2 // NS
CH = 2000
NEG = 0.2
NB = 2048


def _lrelu(v):
    return jnp.where(v >= 0, v, NEG * v)


def _make_edge_kernel(C):
    CPT = C // NS
    NK = E2 // CH
    KPT = NK // NS
    mesh = plsc.VectorSubcoreMesh(
        core_axis_name="c", subcore_axis_name="s", num_cores=NC, num_subcores=NS)

    @functools.partial(
        pl.kernel,
        out_type=[
            jax.ShapeDtypeStruct((NC, C, NP), jnp.float32),
            jax.ShapeDtypeStruct((NC, NS, NP), jnp.float32),
        ],
        mesh=mesh,
        compiler_params=pltpu.CompilerParams(needs_layout_passes=False),
        scratch_types=[
            pltpu.VMEM((NP,), jnp.float32),
            pltpu.VMEM((NP,), jnp.float32),
            pltpu.VMEM((NP,), jnp.float32),
            pltpu.VMEM((CPT, NP), jnp.float32),
            pltpu.VMEM((CPT, NP), jnp.float32),
            pltpu.VMEM((CH,), jnp.int32),
            pltpu.VMEM((CH,), jnp.int32),
            pltpu.VMEM((CH,), jnp.int32),
            pltpu.VMEM((CH,), jnp.int32),
            pltpu.SemaphoreType.DMA,
            pltpu.SemaphoreType.DMA,
        ],
    )
    def edge_kernel(src_h, dst_h, asrc_h, adst_h, hT_h, S_h, den_h,
                    asrc_l, adst_l, den_l, h_l, s_l, src_b0, src_b1,
                    dst_b0, dst_b1, sem0, sem1):
        c = lax.axis_index("c")
        s = lax.axis_index("s")

        pltpu.sync_copy(asrc_h, asrc_l)
        pltpu.sync_copy(adst_h, adst_l)
        pltpu.sync_copy(hT_h.at[pl.ds(s * CPT, CPT)], h_l)

        def zero_all(i, _):
            den_l[pl.ds(i * L, L)] = jnp.zeros((L,), jnp.float32)
            for cc in range(CPT):
                s_l[cc, pl.ds(i * L, L)] = jnp.zeros((L,), jnp.float32)
            return 0
        lax.fori_loop(0, NP // L, zero_all, 0)

        ebase = c * E2
        bufs = ((src_b0, dst_b0, sem0), (src_b1, dst_b1, sem1))

        def fetch(k, b):
            off = ebase + k * CH
            sb, db, sem = bufs[b]
            cp_s = pltpu.make_async_copy(src_h.at[pl.ds(off, CH)], sb, sem)
            cp_d = pltpu.make_async_copy(dst_h.at[pl.ds(off, CH)], db, sem)
            return cp_s, cp_d

        def start_fetch(k, b):
            cp_s, cp_d = fetch(k, b)
            cp_s.start()
            cp_d.start()

        def wait_fetch(k, b):
            cp_s, cp_d = fetch(k, b)
            cp_s.wait()
            cp_d.wait()

        def run_inner(b, with_den):
            sb, db, _ = bufs[b]

            @plsc.parallel_loop(0, CH // L, unroll=8)
            def inner(i):
                sv = sb[pl.ds(i * L, L)]
                dv = db[pl.ds(i * L, L)]
                av = plsc.load_gather(asrc_l, [sv]) + plsc.load_gather(adst_l, [dv])
                wv = jnp.exp(_lrelu(av))
                if with_den:
                    plsc.addupdate_scatter(den_l, [dv], wv)
                for cc in range(CPT):
                    ccv = jnp.full((L,), cc, jnp.int32)
                    hv = plsc.load_gather(h_l, [ccv, sv])
                    plsc.addupdate_scatter(s_l, [ccv, dv], hv * wv)

        def process(k, b):
            mine = jnp.logical_and(k >= s * KPT, k < (s + 1) * KPT)
            lax.cond(
                mine,
                lambda: run_inner(b, True),
                lambda: run_inner(b, False),
            )

        start_fetch(0, 0)

        def chunk_pair(kk, _):
            k0 = 2 * kk
            start_fetch(k0 + 1, 1)
            wait_fetch(k0, 0)
            process(k0, 0)

            @pl.when(k0 + 2 < NK)
            def _():
                start_fetch(k0 + 2, 0)

            wait_fetch(k0 + 1, 1)
            process(k0 + 1, 1)
            return 0
        lax.fori_loop(0, NK // 2, chunk_pair, 0)

        pltpu.sync_copy(den_l, den_h.at[c, s])
        pltpu.sync_copy(s_l, S_h.at[c, pl.ds(s * CPT, CPT)])

    return edge_kernel


_edge_kernel0 = _make_edge_kernel(HID)
_edge_kernel1 = _make_edge_kernel(OUT)


def _tc1_body(xT_ref, w0T_ref, as_ref, ad_ref,
              hT_ref, aso_ref, ado_ref, ws_ref):
    hT = jnp.dot(w0T_ref[...], xT_ref[...], preferred_element_type=jnp.float32)
    hT_ref[...] = hT
    a_s = jnp.dot(as_ref[...], hT, preferred_element_type=jnp.float32)
    a_d = jnp.dot(ad_ref[...], hT, preferred_element_type=jnp.float32)
    aso_ref[...] = a_s
    ado_ref[...] = a_d
    ws_ref[...] = jnp.exp(_lrelu(a_s + a_d))


def _tc2_body(S_ref, den_ref, ws_ref, hT_ref, b0_ref, w1T_ref, as1_ref, ad1_ref,
              h1T_ref, aso_ref, ado_ref, wso_ref):
    ws = ws_ref[...]
    den = jnp.sum(den_ref[...], axis=0, keepdims=True) + ws + 1e-16
    Sb = S_ref[0:HID, :] + S_ref[HID:2 * HID, :] + ws * hT_ref[...]
    x1 = Sb / den + b0_ref[...]
    x1 = jnp.where(x1 > 0, x1, jnp.exp(x1) - 1.0)
    h1T = jnp.dot(w1T_ref[...], x1, preferred_element_type=jnp.float32)
    h1T_ref[...] = h1T
    a_s = jnp.dot(as1_ref[...], h1T, preferred_element_type=jnp.float32)
    a_d = jnp.dot(ad1_ref[...], h1T, preferred_element_type=jnp.float32)
    aso_ref[...] = a_s
    ado_ref[...] = a_d
    wso_ref[...] = jnp.exp(_lrelu(a_s + a_d))


def _tc3_body(S_ref, den_ref, ws_ref, hT_ref, b1_ref, o_ref):
    ws = ws_ref[...]
    den = jnp.sum(den_ref[...], axis=0, keepdims=True) + ws + 1e-16
    Ob = (S_ref[0:OUT, :] + S_ref[OUT:2 * OUT, :] + ws * hT_ref[...]) / den \
        + b1_ref[...]
    m = jnp.max(Ob, axis=0, keepdims=True)
    lse = m + jnp.log(jnp.sum(jnp.exp(Ob - m), axis=0, keepdims=True))
    o_ref[...] = Ob - lse


def _row_spec(rows):
    return pl.BlockSpec((rows, NB), lambda i: (0, i))


def _fix_spec(shape):
    return pl.BlockSpec(shape, lambda i: tuple(0 for _ in shape))


_GRID = NP // NB

_tc1 = pl.pallas_call(
    _tc1_body,
    grid=(_GRID,),
    in_specs=[_row_spec(F_IN), _fix_spec((HID, F_IN)), _fix_spec((1, HID)),
              _fix_spec((1, HID))],
    out_specs=[_row_spec(HID), _row_spec(1), _row_spec(1), _row_spec(1)],
    out_shape=[jax.ShapeDtypeStruct((HID, NP), jnp.float32),
               jax.ShapeDtypeStruct((1, NP), jnp.float32),
               jax.ShapeDtypeStruct((1, NP), jnp.float32),
               jax.ShapeDtypeStruct((1, NP), jnp.float32)],
)

_tc2 = pl.pallas_call(
    _tc2_body,
    grid=(_GRID,),
    in_specs=[_row_spec(2 * HID), _row_spec(NC * NS), _row_spec(1),
              _row_spec(HID), _fix_spec((HID, NB)), _fix_spec((OUT, HID)),
              _fix_spec((1, OUT)), _fix_spec((1, OUT))],
    out_specs=[_row_spec(OUT), _row_spec(1), _row_spec(1), _row_spec(1)],
    out_shape=[jax.ShapeDtypeStruct((OUT, NP), jnp.float32),
               jax.ShapeDtypeStruct((1, NP), jnp.float32),
               jax.ShapeDtypeStruct((1, NP), jnp.float32),
               jax.ShapeDtypeStruct((1, NP), jnp.float32)],
)

_tc3 = pl.pallas_call(
    _tc3_body,
    grid=(_GRID,),
    in_specs=[_row_spec(2 * OUT), _row_spec(NC * NS), _row_spec(1),
              _row_spec(OUT), _fix_spec((OUT, NB))],
    out_specs=_row_spec(OUT),
    out_shape=jax.ShapeDtypeStruct((OUT, NP), jnp.float32),
)


def kernel(x, edge_index, W0, att_src0, att_dst0, b0, W1, att_src1, att_dst1, b1):
    xT = jnp.zeros((F_IN, NP), jnp.float32).at[:, :N].set(x.T)
    src = edge_index[0]
    dst = edge_index[1]
    as0 = att_src0.reshape(1, HID)
    ad0 = att_dst0.reshape(1, HID)
    as1 = att_src1.reshape(1, OUT)
    ad1 = att_dst1.reshape(1, OUT)
    b0b = jnp.broadcast_to(b0[:, None], (HID, NB))
    b1b = jnp.broadcast_to(b1[:, None], (OUT, NB))

    hT0, a_s0, a_d0, ws0 = _tc1(xT, W0.T, as0, ad0)
    S0, den0 = _edge_kernel0(src, dst, a_s0.reshape(NP), a_d0.reshape(NP), hT0)
    h1T, a_s1, a_d1, ws1 = _tc2(
        S0.reshape(NC * HID, NP), den0.reshape(NC * NS, NP), ws0, hT0, b0b,
        W1.T, as1, ad1)
    S1, den1 = _edge_kernel1(src, dst, a_s1.reshape(NP), a_d1.reshape(NP), h1T)
    oT = _tc3(S1.reshape(NC * OUT, NP), den1.reshape(NC * NS, NP), ws1, h1T, b1b)
    return oT.T[:N, :]
